# initial kernel scaffold (unmeasured)
import jax
import jax.numpy as jnp
from jax import lax
from jax.experimental import pallas as pl
from jax.experimental.pallas import tpu as pltpu


def kernel(
    x,
):
    def body(*refs):
        pass

    out_shape = jax.ShapeDtypeStruct(..., jnp.float32)
    return pl.pallas_call(body, out_shape=out_shape)(...)


# baseline (device time: 9583 ns/iter reference)
import jax
import jax.numpy as jnp
from jax import lax
from jax.experimental import pallas as pl
from jax.experimental.pallas import tpu as pltpu

N_DEV = 8


def kernel(x):
    m_per, n = x.shape

    def body(x_ref, out_ref, comm_ref, send_sems, recv_sems):
        my_pos = lax.axis_index("i")

        barrier_sem = pltpu.get_barrier_semaphore()
        for j in range(N_DEV):
            if_peer = j

            @pl.when(if_peer != my_pos)
            def _():
                pl.semaphore_signal(
                    barrier_sem,
                    inc=1,
                    device_id=(if_peer,),
                    device_id_type=pl.DeviceIdType.MESH,
                )

        pl.semaphore_wait(barrier_sem, N_DEV - 1)

        comm_ref[my_pos] = jnp.sum(x_ref[:, :], axis=0, keepdims=True)

        for j in range(N_DEV):
            peer = j

            @pl.when(peer != my_pos)
            def _():
                rdma = pltpu.make_async_remote_copy(
                    src_ref=comm_ref.at[my_pos],
                    dst_ref=comm_ref.at[my_pos],
                    send_sem=send_sems.at[peer],
                    recv_sem=recv_sems.at[my_pos],
                    device_id=(peer,),
                    device_id_type=pl.DeviceIdType.MESH,
                )
                rdma.start()

        for j in range(N_DEV):
            peer = j

            @pl.when(peer != my_pos)
            def _():
                desc = pltpu.make_async_remote_copy(
                    src_ref=comm_ref.at[peer],
                    dst_ref=comm_ref.at[peer],
                    send_sem=send_sems.at[peer],
                    recv_sem=recv_sems.at[peer],
                    device_id=(peer,),
                    device_id_type=pl.DeviceIdType.MESH,
                )
                desc.wait_recv()
                desc.wait_send()

        out_ref[:, :] = jnp.sum(comm_ref[:, 0, :], axis=0, keepdims=True)

    return pl.pallas_call(
        body,
        out_shape=jax.ShapeDtypeStruct((1, n), x.dtype),
        in_specs=[pl.BlockSpec(memory_space=pltpu.VMEM)],
        out_specs=pl.BlockSpec(memory_space=pltpu.VMEM),
        scratch_shapes=[
            pltpu.VMEM((N_DEV, 1, n), x.dtype),
            pltpu.SemaphoreType.DMA((N_DEV,)),
            pltpu.SemaphoreType.DMA((N_DEV,)),
        ],
        compiler_params=pltpu.CompilerParams(collective_id=0),
    )(x)


# device time: 9324 ns/iter; 1.0278x vs baseline; 1.0278x over previous
import jax
import jax.numpy as jnp
from jax import lax
from jax.experimental import pallas as pl
from jax.experimental.pallas import tpu as pltpu

N_DEV = 8


def kernel(x):
    m_per, n = x.shape

    def body(x_ref, out_ref, comm_ref, send_sems, recv_sems):
        my_pos = lax.axis_index("i")

        barrier_sem = pltpu.get_barrier_semaphore()
        for j in range(N_DEV):
            if_peer = j

            @pl.when(if_peer != my_pos)
            def _():
                pl.semaphore_signal(
                    barrier_sem,
                    inc=1,
                    device_id=(if_peer,),
                    device_id_type=pl.DeviceIdType.MESH,
                )

        comm_ref[my_pos] = jnp.sum(x_ref[:, :], axis=0, keepdims=True)

        pl.semaphore_wait(barrier_sem, N_DEV - 1)

        for j in range(N_DEV):
            peer = j

            @pl.when(peer != my_pos)
            def _():
                rdma = pltpu.make_async_remote_copy(
                    src_ref=comm_ref.at[my_pos],
                    dst_ref=comm_ref.at[my_pos],
                    send_sem=send_sems.at[peer],
                    recv_sem=recv_sems.at[my_pos],
                    device_id=(peer,),
                    device_id_type=pl.DeviceIdType.MESH,
                )
                rdma.start()

        for j in range(N_DEV):
            peer = j

            @pl.when(peer != my_pos)
            def _():
                desc = pltpu.make_async_remote_copy(
                    src_ref=comm_ref.at[peer],
                    dst_ref=comm_ref.at[peer],
                    send_sem=send_sems.at[peer],
                    recv_sem=recv_sems.at[peer],
                    device_id=(peer,),
                    device_id_type=pl.DeviceIdType.MESH,
                )
                desc.wait_recv()
                desc.wait_send()

        out_ref[:, :] = jnp.sum(comm_ref[:, 0, :], axis=0, keepdims=True)

    return pl.pallas_call(
        body,
        out_shape=jax.ShapeDtypeStruct((1, n), x.dtype),
        in_specs=[pl.BlockSpec(memory_space=pltpu.VMEM)],
        out_specs=pl.BlockSpec(memory_space=pltpu.VMEM),
        scratch_shapes=[
            pltpu.VMEM((N_DEV, 1, n), x.dtype),
            pltpu.SemaphoreType.DMA((N_DEV,)),
            pltpu.SemaphoreType.DMA((N_DEV,)),
        ],
        compiler_params=pltpu.CompilerParams(collective_id=0),
    )(x)
